# baseline (speedup 1.0000x reference)
import functools

import jax
import jax.numpy as jnp
from jax import lax
from jax.experimental import pallas as pl
from jax.experimental.pallas import tpu as pltpu
from jax.experimental.pallas import tpu_sc as plsc

_GAMMA = 0.5
_TOPK = 512
_NCORE = 2
_NSUB = 16
_NW = _NCORE * _NSUB
_CAP = 1024
_BCAP = 16 * _CAP
_SENT = -3.0e38
_IMIN = -(2 ** 31)
_KEY_LO = -2139095040
_KEY_HI = 2139095039
_SUB_NV = 256
_SUB_RANK = 48
_NUDGE = 1 << 19
_U = 25


def _ceil_avg(lo, hi):
    return (lo >> 1) + (hi >> 1) + (lo & hi & 1) + ((lo ^ hi) & 1)


def _key_to_fvec(mid):
    mv = jnp.full((16,), mid, jnp.int32)
    bv = jnp.where(mv >= 0, mv, _IMIN - mv)
    return plsc.bitcast(bv, jnp.float32)


def _sc_topk(x, tgt):
    b, c = x.shape
    nv = c // 16
    rpw = b // _NW
    mesh = plsc.VectorSubcoreMesh(core_axis_name="c", subcore_axis_name="s")

    @functools.partial(
        pl.kernel,
        out_type=[
            jax.ShapeDtypeStruct((b, _TOPK), jnp.float32),
            jax.ShapeDtypeStruct((b,), jnp.float32),
        ],
        mesh=mesh,
        compiler_params=pltpu.CompilerParams(needs_layout_passes=False,
                                             use_tc_tiling_on_sc=False),
        scratch_types=[
            pltpu.VMEM((c,), jnp.float32),
            pltpu.VMEM((_BCAP + 80,), jnp.float32),
            pltpu.VMEM((_TOPK,), jnp.float32),
            pltpu.VMEM((rpw,), jnp.int32),
            pltpu.VMEM((rpw,), jnp.float32),
            pltpu.VMEM((16,), jnp.int32),
            pltpu.VMEM((16,), jnp.int32),
            pltpu.VMEM((16,), jnp.int32),
            pltpu.SemaphoreType.DMA,
            pltpu.SemaphoreType.DMA,
        ],
    )
    def sc_kernel(x_hbm, tgt_hbm, outneg_hbm, outpos_hbm,
                  row_v, blk_v, out_v, tgt_v, pos_v, cnt_v, key_v, nw_v,
                  dma_sem, out_sem):
        wid = lax.axis_index("s") * _NCORE + lax.axis_index("c")
        base = wid * rpw
        iota = lax.iota(jnp.int32, 16)
        lane0 = iota == 0
        zero16 = jnp.zeros((16,), jnp.int32)
        sent_vec = jnp.full((16,), _SENT, jnp.float32)

        def count_ref(ref, ngroups, tvec):
            def cbody(i, acc):
                ps = []
                for u in range(4):
                    xv = ref[pl.ds((i * 4 + u) * 16, 16)]
                    ps.append(plsc.all_reduce_population_count(xv >= tvec))
                return acc + ((ps[0] + ps[1]) + (ps[2] + ps[3]))
            return jnp.max(lax.fori_loop(0, ngroups, cbody, zero16))

        def seg_sweep(tvec, base0, niters, off0):
            def sbody(it, off):
                xs, advs = [], []
                for u in range(_U):
                    xv = row_v[pl.ds((base0 + it * _U + u) * 16, 16)]
                    m = xv >= tvec
                    pc = plsc.all_reduce_population_count(m)
                    xs.append(xv)
                    advs.append(jnp.where(pc > 0, jnp.int32(16), jnp.int32(0)))
                for u in range(_U):
                    idx = (off & jnp.int32(_BCAP - 1)) + iota
                    plsc.store_scatter(blk_v, [idx], xs[u])
                    off = off + advs[u]
                return off
            return lax.fori_loop(0, niters, sbody, off0)

        def exact_compact(tvec, nwords):
            def p2(k, carry):
                off2, mx = carry
                xs, ms, css, pcs = [], [], [], []
                for u in range(4):
                    xv = blk_v[pl.ds((k * 4 + u) * 16, 16)]
                    m = xv >= tvec
                    css.append(plsc.cumsum(
                        jnp.where(m, jnp.int32(1), jnp.int32(0))))
                    pcs.append(plsc.all_reduce_population_count(m))
                    mx = jnp.maximum(mx, jnp.where(m, xv, sent_vec))
                    xs.append(xv)
                    ms.append(m)
                for u in range(4):
                    plsc.store_scatter(blk_v, [off2 + css[u] - 1], xs[u],
                                       mask=ms[u])
                    off2 = off2 + pcs[u]
                return off2, mx
            return lax.fori_loop(0, (nwords // 16 + 3) // 4, p2,
                                 (zero16, sent_vec))

        def strict_compact(tvec):
            def sbody(i, off):
                xv = row_v[pl.ds(i * 16, 16)]
                m = xv > tvec
                cs = plsc.cumsum(jnp.where(m, jnp.int32(1), jnp.int32(0)))
                idx = jnp.minimum(off + cs - 1, jnp.int32(_BCAP + 15))
                plsc.store_scatter(blk_v, [idx], xv, mask=m)
                return off + plsc.all_reduce_population_count(m)
            return lax.fori_loop(0, nv, sbody, zero16)

        def bootstrap():
            def cond(st):
                lo, hi = st
                return lo < hi
            def bbody(st):
                lo, hi = st
                mid = _ceil_avg(lo, hi)
                cc = count_ref(row_v, _SUB_NV // 4, _key_to_fvec(mid))
                return (jnp.where(cc >= _SUB_RANK, mid, lo),
                        jnp.where(cc >= _SUB_RANK, hi, mid - 1))
            lo, _ = lax.while_loop(
                cond, bbody, (jnp.int32(_KEY_LO), jnp.int32(_KEY_HI)))
            return lo

        def process_row(r, warm, fetch_next):
            ch = c // 2
            nvh = (nv // 2) // _U

            def prefetch_half(w):
                pltpu.async_copy(x_hbm.at[r + 1].at[pl.ds(w * ch, ch)],
                                 row_v.at[pl.ds(w * ch, ch)], dma_sem)

            def drain_and_restore_half0():
                pltpu.make_async_copy(
                    x_hbm.at[r + 1].at[pl.ds(0, ch)],
                    row_v.at[pl.ds(0, ch)], dma_sem).wait()
                pltpu.sync_copy(x_hbm.at[r].at[pl.ds(0, ch)],
                                row_v.at[pl.ds(0, ch)])

            def cond(st):
                lo, hi, mid, it, hit = st
                return jnp.logical_and(jnp.logical_not(hit), lo < hi)

            def obody(st):
                lo, hi, mid, it, _ = st
                tvec = _key_to_fvec(mid)

                @pl.when(jnp.logical_and(it == 1, fetch_next))
                def _():
                    drain_and_restore_half0()

                nw_v[...] = zero16

                @pl.when(it == 0)
                def _():
                    offa = seg_sweep(tvec, 0, nvh, zero16)

                    @pl.when(fetch_next)
                    def _():
                        prefetch_half(0)

                    nw_v[...] = seg_sweep(tvec, nvh * _U, nvh, offa)

                @pl.when(it != 0)
                def _():
                    nw_v[...] = seg_sweep(tvec, 0, nv // _U, zero16)

                nw = jnp.max(nw_v[...])
                cap_ok = nw <= _BCAP
                cnt_v[...] = zero16

                @pl.when(cap_ok)
                def _():
                    for w in range(4):
                        plsc.store_scatter(
                            blk_v,
                            [jnp.full((16,), nw + 16 * w, jnp.int32) + iota],
                            sent_vec)
                    c2v, mxv = exact_compact(tvec, nw)
                    cnt_v[...] = c2v
                    bk = plsc.bitcast(mxv, jnp.int32)
                    kk = jnp.where(bk >= 0, bk, _IMIN - bk)
                    key_v[...] = jnp.minimum(jnp.full((16,), hi, jnp.int32),
                                             kk)

                c2 = jnp.max(cnt_v[...])
                ge = jnp.logical_or(jnp.logical_not(cap_ok), c2 >= _TOPK)
                hit = jnp.logical_and(cap_ok, c2 >= _TOPK)
                lo2 = jnp.where(ge, mid, lo)
                hi2 = jnp.where(ge, hi, mid - 1)
                mid2 = jnp.where(hit, mid, _ceil_avg(lo2, hi2))
                return (lo2, hi2, mid2, it + 1, hit)

            mid0 = jnp.clip(warm, jnp.int32(_KEY_LO + 1), jnp.int32(_KEY_HI))
            st0 = (jnp.int32(_KEY_LO), jnp.int32(_KEY_HI), mid0,
                   jnp.int32(0), jnp.bool_(False))
            lo, hi, mid, it, hit = lax.while_loop(cond, obody, st0)

            a_live = jnp.logical_and(hit, it == 1)

            @pl.when(jnp.logical_not(hit))
            def _():
                @pl.when(jnp.logical_and(it == 1, fetch_next))
                def _():
                    drain_and_restore_half0()
                cnt_v[...] = strict_compact(_key_to_fvec(lo))
                key_v[...] = jnp.full((16,), lo, jnp.int32)

            @pl.when(jnp.logical_and(fetch_next, a_live))
            def _():
                prefetch_half(1)

            @pl.when(jnp.logical_and(fetch_next, jnp.logical_not(a_live)))
            def _():
                prefetch_half(0)
                prefetch_half(1)

            c2 = jnp.max(cnt_v[...])
            hi2 = jnp.max(key_v[...])
            lo2 = jnp.where(hit, mid, lo)

            for w in range(4):
                plsc.store_scatter(
                    blk_v,
                    [jnp.full((16,), c2 + 16 * w, jnp.int32) + iota],
                    sent_vec)
            ng2 = (c2 + 63) // 64
            nv2 = (c2 + 15) // 16

            def rcond(st):
                rlo, rhi = st
                return rlo < rhi
            def rbody(st):
                rlo, rhi = st
                rmid = _ceil_avg(rlo, rhi)
                rc = count_ref(blk_v, ng2, _key_to_fvec(rmid))
                return (jnp.where(rc >= _TOPK, rmid, rlo),
                        jnp.where(rc >= _TOPK, rhi, rmid - 1))
            t_key, _ = lax.while_loop(rcond, rbody, (lo2, hi2))

            @pl.when(r > base)
            def _():
                pltpu.make_async_copy(out_v, outneg_hbm.at[r], out_sem).wait()

            tvec = _key_to_fvec(t_key)
            def fbody(v, _):
                out_v[pl.ds(v * 16, 16)] = tvec
                return 0
            lax.fori_loop(0, _TOPK // 16, fbody, 0)

            def ebody(i, off):
                xv = blk_v[pl.ds(i * 16, 16)]
                m = xv > tvec
                cs = plsc.cumsum(jnp.where(m, jnp.int32(1), jnp.int32(0)))
                idx = jnp.minimum(off + cs - 1, jnp.int32(_TOPK - 1))
                plsc.store_scatter(out_v, [idx], xv, mask=m)
                return off + plsc.all_reduce_population_count(m)
            lax.fori_loop(0, nv2, ebody, zero16)

            pltpu.async_copy(out_v, outneg_hbm.at[r], out_sem)
            return t_key

        def prep_row(j):
            jv = jnp.full((16,), j, jnp.int32)
            tg = plsc.load_gather(tgt_v, [jv])
            xpos = plsc.load_gather(row_v, [tg])
            plsc.store_scatter(pos_v, [jv], xpos, mask=lane0)
            plsc.store_scatter(row_v, [tg], sent_vec, mask=lane0)

        pltpu.sync_copy(tgt_hbm.at[pl.ds(base, rpw)], tgt_v)

        pltpu.sync_copy(x_hbm.at[base], row_v)
        prep_row(jnp.int32(0))
        t0 = process_row(base, bootstrap(), jnp.bool_(rpw > 1))

        def rowbody(j, warm):
            r = base + j
            pltpu.make_async_copy(x_hbm.at[r], row_v, dma_sem).wait()
            prep_row(j)
            t = process_row(r, warm, j < rpw - 1)
            return t - _NUDGE

        lax.fori_loop(1, rpw, rowbody, t0 - _NUDGE)
        pltpu.make_async_copy(out_v, outneg_hbm.at[base], out_sem).wait()
        pltpu.sync_copy(pos_v, outpos_hbm.at[pl.ds(base, rpw)])

    return sc_kernel(x, tgt)


def _f_neg(v):
    s = 1.0 / (1.0 + jnp.exp(-v))
    return jnp.log(1.0 - jnp.minimum(s, jnp.float32(0.999)))


def _tc_reduce_body(neg_ref, pos_ref, out_ref):
    neg_sum = jnp.sum(_f_neg(neg_ref[...]), keepdims=True)
    p = jnp.maximum(1.0 / (1.0 + jnp.exp(-pos_ref[...])), jnp.float32(0.001))
    pos_sum = jnp.sum(jnp.log(p), keepdims=True)
    b = pos_ref.shape[0] * pos_ref.shape[1]
    out_ref[...] = (_GAMMA * (-pos_sum / b)
                    + (1.0 - _GAMMA) * (-neg_sum / (b * _TOPK)))


def kernel(input, target):
    b, c = input.shape
    neg_tops, pos_logits = _sc_topk(input, target.astype(jnp.int32))
    loss = pl.pallas_call(
        _tc_reduce_body,
        out_shape=jax.ShapeDtypeStruct((1, 1), jnp.float32),
    )(neg_tops, pos_logits.reshape(8, b // 8))
    return loss[0, 0]


# speedup vs baseline: 1.4281x; 1.4281x over previous
import functools

import jax
import jax.numpy as jnp
from jax import lax
from jax.experimental import pallas as pl
from jax.experimental.pallas import tpu as pltpu
from jax.experimental.pallas import tpu_sc as plsc

_GAMMA = 0.5
_TOPK = 512
_NCORE = 2
_NSUB = 16
_NW = _NCORE * _NSUB
_CAP = 1024
_BCAP = 16 * _CAP
_SENT = -3.0e38
_IMIN = -(2 ** 31)
_KEY_LO = -2139095040
_KEY_HI = 2139095039
_SUB_NV = 256
_SUB_RANK = 48
_NUDGE = 1 << 19
_U = 25


def _ceil_avg(lo, hi):
    return (lo >> 1) + (hi >> 1) + (lo & hi & 1) + ((lo ^ hi) & 1)


def _key_to_fvec(mid):
    mv = jnp.full((16,), mid, jnp.int32)
    bv = jnp.where(mv >= 0, mv, _IMIN - mv)
    return plsc.bitcast(bv, jnp.float32)


def _sc_topk(x, tgt):
    b, c = x.shape
    nv = c // 16
    rpw = b // _NW
    mesh = plsc.VectorSubcoreMesh(core_axis_name="c", subcore_axis_name="s")

    @functools.partial(
        pl.kernel,
        out_type=[
            jax.ShapeDtypeStruct((b, _TOPK), jnp.float32),
            jax.ShapeDtypeStruct((b,), jnp.float32),
        ],
        mesh=mesh,
        compiler_params=pltpu.CompilerParams(needs_layout_passes=False),
        scratch_types=[
            pltpu.VMEM((c,), jnp.float32),
            pltpu.VMEM((_BCAP + 80,), jnp.float32),
            pltpu.VMEM((_TOPK,), jnp.float32),
            pltpu.VMEM((rpw,), jnp.int32),
            pltpu.VMEM((rpw,), jnp.float32),
            pltpu.VMEM((16,), jnp.int32),
            pltpu.VMEM((16,), jnp.int32),
            pltpu.SemaphoreType.DMA,
            pltpu.SemaphoreType.DMA,
        ],
    )
    def sc_kernel(x_hbm, tgt_hbm, outneg_hbm, outpos_hbm,
                  row_v, blk_v, out_v, tgt_v, pos_v, cnt_v, key_v,
                  dma_sem, out_sem):
        wid = lax.axis_index("s") * _NCORE + lax.axis_index("c")
        base = wid * rpw
        iota = lax.iota(jnp.int32, 16)
        lane0 = iota == 0
        zero16 = jnp.zeros((16,), jnp.int32)
        sent_vec = jnp.full((16,), _SENT, jnp.float32)

        def count_ref(ref, ngroups, tvec):
            def cbody(i, acc):
                ps = []
                for u in range(4):
                    xv = ref[pl.ds((i * 4 + u) * 16, 16)]
                    ps.append(plsc.all_reduce_population_count(xv >= tvec))
                return acc + ((ps[0] + ps[1]) + (ps[2] + ps[3]))
            return jnp.max(lax.fori_loop(0, ngroups, cbody, zero16))

        def seg_sweep(tvec, base0, niters, off0):
            def sbody(it, off):
                xs, advs = [], []
                for u in range(_U):
                    xv = row_v[pl.ds((base0 + it * _U + u) * 16, 16)]
                    m = xv >= tvec
                    pc = plsc.all_reduce_population_count(m)
                    xs.append(xv)
                    advs.append(jnp.where(pc > 0, jnp.int32(16), jnp.int32(0)))
                for u in range(_U):
                    idx = (off & jnp.int32(_BCAP - 1)) + iota
                    plsc.store_scatter(blk_v, [idx], xs[u])
                    off = off + advs[u]
                return off
            return lax.fori_loop(0, niters, sbody, off0)

        def exact_compact(tvec, nwords):
            def p2(k, carry):
                off2, mx = carry
                xs, ms, css, pcs = [], [], [], []
                for u in range(4):
                    xv = blk_v[pl.ds((k * 4 + u) * 16, 16)]
                    m = xv >= tvec
                    css.append(plsc.cumsum(
                        jnp.where(m, jnp.int32(1), jnp.int32(0))))
                    pcs.append(plsc.all_reduce_population_count(m))
                    mx = jnp.maximum(mx, jnp.where(m, xv, sent_vec))
                    xs.append(xv)
                    ms.append(m)
                for u in range(4):
                    plsc.store_scatter(blk_v, [off2 + css[u] - 1], xs[u],
                                       mask=ms[u])
                    off2 = off2 + pcs[u]
                return off2, mx
            return lax.fori_loop(0, (nwords // 16 + 3) // 4, p2,
                                 (zero16, sent_vec))

        def strict_compact(tvec):
            def sbody(i, off):
                xv = row_v[pl.ds(i * 16, 16)]
                m = xv > tvec
                cs = plsc.cumsum(jnp.where(m, jnp.int32(1), jnp.int32(0)))
                idx = jnp.minimum(off + cs - 1, jnp.int32(_BCAP + 15))
                plsc.store_scatter(blk_v, [idx], xv, mask=m)
                return off + plsc.all_reduce_population_count(m)
            return lax.fori_loop(0, nv, sbody, zero16)

        def bootstrap():
            def cond(st):
                lo, hi = st
                return lo < hi
            def bbody(st):
                lo, hi = st
                mid = _ceil_avg(lo, hi)
                cc = count_ref(row_v, _SUB_NV // 4, _key_to_fvec(mid))
                return (jnp.where(cc >= _SUB_RANK, mid, lo),
                        jnp.where(cc >= _SUB_RANK, hi, mid - 1))
            lo, _ = lax.while_loop(
                cond, bbody, (jnp.int32(_KEY_LO), jnp.int32(_KEY_HI)))
            return lo

        def process_row(r, warm, fetch_next):
            def cond(st):
                lo, hi, mid, it, hit = st
                return jnp.logical_and(jnp.logical_not(hit), lo < hi)

            def obody(st):
                lo, hi, mid, it, _ = st
                tvec = _key_to_fvec(mid)
                nw = jnp.max(seg_sweep(tvec, 0, nv // _U, zero16))
                cap_ok = nw <= _BCAP
                cnt_v[...] = zero16

                @pl.when(cap_ok)
                def _():
                    for w in range(4):
                        plsc.store_scatter(
                            blk_v,
                            [jnp.full((16,), nw + 16 * w, jnp.int32) + iota],
                            sent_vec)
                    c2v, mxv = exact_compact(tvec, nw)
                    cnt_v[...] = c2v
                    bk = plsc.bitcast(mxv, jnp.int32)
                    kk = jnp.where(bk >= 0, bk, _IMIN - bk)
                    key_v[...] = jnp.minimum(jnp.full((16,), hi, jnp.int32),
                                             kk)

                c2 = jnp.max(cnt_v[...])
                ge = jnp.logical_or(jnp.logical_not(cap_ok), c2 >= _TOPK)
                hit = jnp.logical_and(cap_ok, c2 >= _TOPK)
                lo2 = jnp.where(ge, mid, lo)
                hi2 = jnp.where(ge, hi, mid - 1)
                mid2 = jnp.where(hit, mid, _ceil_avg(lo2, hi2))
                return (lo2, hi2, mid2, it + 1, hit)

            mid0 = jnp.clip(warm, jnp.int32(_KEY_LO + 1), jnp.int32(_KEY_HI))
            st0 = (jnp.int32(_KEY_LO), jnp.int32(_KEY_HI), mid0,
                   jnp.int32(0), jnp.bool_(False))
            lo, hi, mid, it, hit = lax.while_loop(cond, obody, st0)

            @pl.when(jnp.logical_not(hit))
            def _():
                cnt_v[...] = strict_compact(_key_to_fvec(lo))
                key_v[...] = jnp.full((16,), lo, jnp.int32)

            @pl.when(fetch_next)
            def _():
                pltpu.async_copy(x_hbm.at[r + 1], row_v, dma_sem)

            c2 = jnp.max(cnt_v[...])
            hi2 = jnp.max(key_v[...])
            lo2 = jnp.where(hit, mid, lo)

            for w in range(4):
                plsc.store_scatter(
                    blk_v,
                    [jnp.full((16,), c2 + 16 * w, jnp.int32) + iota],
                    sent_vec)
            ng2 = (c2 + 63) // 64
            nv2 = (c2 + 15) // 16

            def rcond(st):
                rlo, rhi = st
                return rlo < rhi
            def rbody(st):
                rlo, rhi = st
                rmid = _ceil_avg(rlo, rhi)
                rc = count_ref(blk_v, ng2, _key_to_fvec(rmid))
                return (jnp.where(rc >= _TOPK, rmid, rlo),
                        jnp.where(rc >= _TOPK, rhi, rmid - 1))
            t_key, _ = lax.while_loop(rcond, rbody, (lo2, hi2))

            @pl.when(r > base)
            def _():
                pltpu.make_async_copy(out_v, outneg_hbm.at[r], out_sem).wait()

            tvec = _key_to_fvec(t_key)
            def fbody(v, _):
                out_v[pl.ds(v * 16, 16)] = tvec
                return 0
            lax.fori_loop(0, _TOPK // 16, fbody, 0)

            def ebody(i, off):
                xv = blk_v[pl.ds(i * 16, 16)]
                m = xv > tvec
                cs = plsc.cumsum(jnp.where(m, jnp.int32(1), jnp.int32(0)))
                idx = jnp.minimum(off + cs - 1, jnp.int32(_TOPK - 1))
                plsc.store_scatter(out_v, [idx], xv, mask=m)
                return off + plsc.all_reduce_population_count(m)
            lax.fori_loop(0, nv2, ebody, zero16)

            pltpu.async_copy(out_v, outneg_hbm.at[r], out_sem)
            return t_key

        def prep_row(j):
            jv = jnp.full((16,), j, jnp.int32)
            tg = plsc.load_gather(tgt_v, [jv])
            xpos = plsc.load_gather(row_v, [tg])
            plsc.store_scatter(pos_v, [jv], xpos, mask=lane0)
            plsc.store_scatter(row_v, [tg], sent_vec, mask=lane0)

        pltpu.sync_copy(tgt_hbm.at[pl.ds(base, rpw)], tgt_v)

        pltpu.sync_copy(x_hbm.at[base], row_v)
        prep_row(jnp.int32(0))
        t0 = process_row(base, bootstrap(), jnp.bool_(rpw > 1))

        def rowbody(j, warm):
            r = base + j
            pltpu.make_async_copy(x_hbm.at[r], row_v, dma_sem).wait()
            prep_row(j)
            t = process_row(r, warm, j < rpw - 1)
            return t - _NUDGE

        lax.fori_loop(1, rpw, rowbody, t0 - _NUDGE)
        pltpu.make_async_copy(out_v, outneg_hbm.at[base], out_sem).wait()
        pltpu.sync_copy(pos_v, outpos_hbm.at[pl.ds(base, rpw)])

    return sc_kernel(x, tgt)


def _f_neg(v):
    s = 1.0 / (1.0 + jnp.exp(-v))
    return jnp.log(1.0 - jnp.minimum(s, jnp.float32(0.999)))


def _tc_reduce_body(neg_ref, pos_ref, out_ref):
    neg_sum = jnp.sum(_f_neg(neg_ref[...]), keepdims=True)
    p = jnp.maximum(1.0 / (1.0 + jnp.exp(-pos_ref[...])), jnp.float32(0.001))
    pos_sum = jnp.sum(jnp.log(p), keepdims=True)
    b = pos_ref.shape[0] * pos_ref.shape[1]
    out_ref[...] = (_GAMMA * (-pos_sum / b)
                    + (1.0 - _GAMMA) * (-neg_sum / (b * _TOPK)))


def kernel(input, target):
    b, c = input.shape
    neg_tops, pos_logits = _sc_topk(input, target.astype(jnp.int32))
    loss = pl.pallas_call(
        _tc_reduce_body,
        out_shape=jax.ShapeDtypeStruct((1, 1), jnp.float32),
    )(neg_tops, pos_logits.reshape(8, b // 8))
    return loss[0, 0]
